# Initial kernel scaffold; baseline (speedup 1.0000x reference)
import jax
import jax.numpy as jnp
from jax.experimental import pallas as pl


def kernel(x, edge_index, edge_feature, W_msg_node, b_msg_node, W_msg_edge, b_msg_edge, W_gate_node, b_gate_node, W_gate_edge, b_gate_edge, bias, W_self, b_self):
    raise NotImplementedError("write your pallas kernel here")


# speedup vs baseline: 1.0550x; 1.0550x over previous
import functools

import jax
import jax.numpy as jnp
from jax import lax
from jax.experimental import pallas as pl
from jax.experimental.pallas import tpu as pltpu
from jax.experimental.pallas import tpu_sc as plsc

_NC = 2
_NS = 16


def _mg_body(x_ref, wm_ref, wg_ref, bm_ref, bg_ref, mg_ref):
    d = x_ref.shape[1]
    xb = x_ref[...]
    mg_ref[:, :d] = jnp.dot(xb, wm_ref[...], preferred_element_type=jnp.float32) + bm_ref[...]
    mg_ref[:, d:] = jnp.dot(xb, wg_ref[...], preferred_element_type=jnp.float32) + bg_ref[...]


def _tc_mg(x, w_msg, w_gate, bm, bg):
    n, d = x.shape
    br = 1000
    return pl.pallas_call(
        _mg_body,
        grid=(n // br,),
        in_specs=[
            pl.BlockSpec((br, d), lambda i: (i, 0)),
            pl.BlockSpec((d, d), lambda i: (0, 0)),
            pl.BlockSpec((d, d), lambda i: (0, 0)),
            pl.BlockSpec((1, d), lambda i: (0, 0)),
            pl.BlockSpec((1, d), lambda i: (0, 0)),
        ],
        out_specs=pl.BlockSpec((br, 2 * d), lambda i: (i, 0)),
        out_shape=jax.ShapeDtypeStruct((n, 2 * d), jnp.float32),
    )(x, w_msg, w_gate, bm, bg)


def _fin_body(p_ref, x_ref, ws_ref, b_ref, o_ref):
    s = jnp.dot(x_ref[...], ws_ref[...], preferred_element_type=jnp.float32)
    o_ref[...] = jnp.maximum(p_ref[0] + p_ref[1] + s + b_ref[...], 0.0)


def _tc_fin(parts, x, w_self, bf):
    n, d = x.shape
    br = 1000
    return pl.pallas_call(
        _fin_body,
        grid=(n // br,),
        in_specs=[
            pl.BlockSpec((2, br, d), lambda i: (0, i, 0)),
            pl.BlockSpec((br, d), lambda i: (i, 0)),
            pl.BlockSpec((d, d), lambda i: (0, 0)),
            pl.BlockSpec((1, d), lambda i: (0, 0)),
        ],
        out_specs=pl.BlockSpec((br, d), lambda i: (i, 0)),
        out_shape=jax.ShapeDtypeStruct((n, d), jnp.float32),
    )(parts, x, w_self, bf)


@functools.lru_cache(maxsize=None)
def _make_sc_edge(n, d, e, k, zr):
    nw = _NC * _NS
    epw = e // nw
    ch = epw // k
    rpt = -(-n // (_NS * zr)) * zr
    npad = rpt * _NS
    zc = rpt // zr
    assert epw * nw == e and ch * k == epw
    assert k % 8 == 0 and k <= 128 and zr % 8 == 0
    mesh = plsc.VectorSubcoreMesh(core_axis_name="c", subcore_axis_name="s")

    @functools.partial(
        pl.kernel,
        mesh=mesh,
        compiler_params=pltpu.CompilerParams(needs_layout_passes=False),
        out_type=jax.ShapeDtypeStruct((_NC, npad, d), jnp.float32),
        scratch_types=[
            pltpu.VMEM((k,), jnp.int32),
            pltpu.VMEM((k,), jnp.int32),
            pltpu.VMEM((k,), jnp.float32),
            pltpu.VMEM((k, 2 * d), jnp.float32),
            pltpu.VMEM((k, d), jnp.float32),
            pltpu.VMEM((2 * d,), jnp.float32),
            pltpu.VMEM((zr, d), jnp.float32),
            pltpu.VMEM_SHARED((npad, d), jnp.float32),
            pltpu.SemaphoreType.DMA,
        ],
    )
    def sc_edge(mg_hbm, src_hbm, dst_hbm, ef_hbm, wmg_hbm, out_hbm,
                src_v, dst_v, ef_v, mg_v, r_v, wmg_v, z_v, acc, sem):
        cid = lax.axis_index("c")
        sid = lax.axis_index("s")
        wid = sid * _NC + cid
        zeros16 = jnp.zeros((16,), jnp.float32)

        def zrow(i, carry):
            for j in range(d // 16):
                z_v[i, pl.ds(j * 16, 16)] = zeros16
            return carry

        lax.fori_loop(0, zr, zrow, 0)
        row0 = sid * rpt

        def zcopy(t, carry):
            pltpu.sync_copy(z_v, acc.at[pl.ds(row0 + t * zr, zr)])
            return carry

        lax.fori_loop(0, zc, zcopy, 0)
        pltpu.sync_copy(wmg_hbm, wmg_v)
        plsc.subcore_barrier()

        base = wid * epw

        def chunk(c, carry):
            e0 = base + c * k
            pltpu.sync_copy(src_hbm.at[pl.ds(e0, k)], src_v)
            pltpu.sync_copy(dst_hbm.at[pl.ds(e0, k)], dst_v)
            pltpu.sync_copy(ef_hbm.at[pl.ds(e0, k)], ef_v)
            pltpu.async_copy(mg_hbm.at[src_v], mg_v, sem).wait()

            def edge(i, icarry):
                efk = plsc.load_gather(ef_v, [lax.broadcast(i, (16,))])
                for j in range(d // 16):
                    slm = pl.ds(j * 16, 16)
                    slg = pl.ds(d + j * 16, 16)
                    m = mg_v[i, slm] + efk * wmg_v[slm]
                    g = mg_v[i, slg] + efk * wmg_v[slg]
                    r_v[i, slm] = m / (1.0 + jnp.exp(-g))
                return icarry

            lax.fori_loop(0, k, edge, 0)
            pltpu.sync_copy(r_v, acc.at[dst_v], add=True)
            return carry

        lax.fori_loop(0, ch, chunk, 0)
        plsc.subcore_barrier()

        def ocopy(t, carry):
            r0 = sid * rpt + t * zr
            pltpu.sync_copy(acc.at[pl.ds(r0, zr)], z_v)
            pltpu.sync_copy(z_v, out_hbm.at[cid, pl.ds(r0, zr)])
            return carry

        lax.fori_loop(0, zc, ocopy, 0)

    return sc_edge


def kernel(x, edge_index, edge_feature, W_msg_node, b_msg_node, W_msg_edge, b_msg_edge,
           W_gate_node, b_gate_node, W_gate_edge, b_gate_edge, bias, W_self, b_self):
    n, d = x.shape
    e = edge_index.shape[1]
    src = edge_index[0].astype(jnp.int32)
    dst = edge_index[1].astype(jnp.int32)
    ef = edge_feature[:, 0]
    bm = (b_msg_node + b_msg_edge).reshape(1, d)
    bg = (b_gate_node + b_gate_edge).reshape(1, d)
    wmg = jnp.concatenate([W_msg_edge[0], W_gate_edge[0]])
    mg = _tc_mg(x, W_msg_node, W_gate_node, bm, bg)
    parts = _make_sc_edge(n, d, e, 80, 128)(mg, src, dst, ef, wmg)
    bf = (b_self + bias).reshape(1, d)
    return _tc_fin(parts, x, W_self, bf)


# speedup vs baseline: 1.2538x; 1.1884x over previous
import functools

import jax
import jax.numpy as jnp
from jax import lax
from jax.experimental import pallas as pl
from jax.experimental.pallas import tpu as pltpu
from jax.experimental.pallas import tpu_sc as plsc

_NC = 2
_NS = 16
_PIPELINED = True


def _mg_body(x_ref, wm_ref, wg_ref, bm_ref, bg_ref, mg_ref):
    d = x_ref.shape[1]
    xb = x_ref[...]
    mg_ref[:, :d] = jnp.dot(xb, wm_ref[...], preferred_element_type=jnp.float32) + bm_ref[...]
    mg_ref[:, d:] = jnp.dot(xb, wg_ref[...], preferred_element_type=jnp.float32) + bg_ref[...]


def _tc_mg(x, w_msg, w_gate, bm, bg):
    n, d = x.shape
    br = 1000
    return pl.pallas_call(
        _mg_body,
        grid=(n // br,),
        in_specs=[
            pl.BlockSpec((br, d), lambda i: (i, 0)),
            pl.BlockSpec((d, d), lambda i: (0, 0)),
            pl.BlockSpec((d, d), lambda i: (0, 0)),
            pl.BlockSpec((1, d), lambda i: (0, 0)),
            pl.BlockSpec((1, d), lambda i: (0, 0)),
        ],
        out_specs=pl.BlockSpec((br, 2 * d), lambda i: (i, 0)),
        out_shape=jax.ShapeDtypeStruct((n, 2 * d), jnp.float32),
    )(x, w_msg, w_gate, bm, bg)


def _fin_body(p_ref, x_ref, ws_ref, b_ref, o_ref):
    s = jnp.dot(x_ref[...], ws_ref[...], preferred_element_type=jnp.float32)
    o_ref[...] = jnp.maximum(p_ref[0] + p_ref[1] + s + b_ref[...], 0.0)


def _tc_fin(parts, x, w_self, bf):
    n, d = x.shape
    br = 1000
    return pl.pallas_call(
        _fin_body,
        grid=(n // br,),
        in_specs=[
            pl.BlockSpec((2, br, d), lambda i: (0, i, 0)),
            pl.BlockSpec((br, d), lambda i: (i, 0)),
            pl.BlockSpec((d, d), lambda i: (0, 0)),
            pl.BlockSpec((1, d), lambda i: (0, 0)),
        ],
        out_specs=pl.BlockSpec((br, d), lambda i: (i, 0)),
        out_shape=jax.ShapeDtypeStruct((n, d), jnp.float32),
    )(parts, x, w_self, bf)


@functools.lru_cache(maxsize=None)
def _make_sc_edge(n, d, e, k):
    nw = _NC * _NS
    epw = e // nw
    ch = epw // k
    rpt = -(-n // (_NS * k)) * k
    npad = rpt * _NS
    zc = rpt // k
    assert epw * nw == e and ch * k == epw and ch % 2 == 0 and ch >= 6
    assert k % 8 == 0 and k <= 128
    nj = d // 16
    pairs = ch // 2
    mesh = plsc.VectorSubcoreMesh(core_axis_name="c", subcore_axis_name="s")

    @functools.partial(
        pl.kernel,
        mesh=mesh,
        compiler_params=pltpu.CompilerParams(needs_layout_passes=False),
        out_type=jax.ShapeDtypeStruct((_NC, npad, d), jnp.float32),
        scratch_types=[
            pltpu.VMEM((k,), jnp.int32),
            pltpu.VMEM((k,), jnp.int32),
            pltpu.VMEM((k,), jnp.float32),
            pltpu.VMEM((k,), jnp.float32),
            pltpu.VMEM((k,), jnp.int32),
            pltpu.VMEM((k,), jnp.int32),
            pltpu.VMEM((k, 2 * d), jnp.float32),
            pltpu.VMEM((k, 2 * d), jnp.float32),
            pltpu.VMEM((k, d), jnp.float32),
            pltpu.VMEM((k, d), jnp.float32),
            pltpu.VMEM((2 * d,), jnp.float32),
            pltpu.VMEM_SHARED((npad, d), jnp.float32),
            [pltpu.SemaphoreType.DMA] * 8,
        ],
    )
    def sc_edge(mg_hbm, src_hbm, dst_hbm, ef_hbm, wmg_hbm, out_hbm,
                s_a, s_b, e_a, e_b, d_a, d_b, mg_a, mg_b, r_a, r_b,
                wmg_v, acc, sems):
        gsem_a, gsem_b, ssem_a, ssem_b, isem_a, isem_b, dsem_a, dsem_b = sems
        cid = lax.axis_index("c")
        sid = lax.axis_index("s")
        wid = sid * _NC + cid
        zeros16 = jnp.zeros((16,), jnp.float32)

        def zrow(i, carry):
            for j in range(nj):
                r_a[i, pl.ds(j * 16, 16)] = zeros16
            return carry

        lax.fori_loop(0, k, zrow, 0)
        row0 = sid * rpt

        def zcopy(t, carry):
            pltpu.sync_copy(r_a, acc.at[pl.ds(row0 + t * k, k)])
            return carry

        lax.fori_loop(0, zc, zcopy, 0)

        pltpu.sync_copy(wmg_hbm, wmg_v)
        wm = [wmg_v[pl.ds(j * 16, 16)] for j in range(nj)]
        wg = [wmg_v[pl.ds(d + j * 16, 16)] for j in range(nj)]
        plsc.subcore_barrier()

        base = wid * epw

        def compute(ev, mg, r):
            def edge(i, carry):
                efk = plsc.load_gather(ev, [lax.broadcast(i, (16,))])
                for j in range(nj):
                    m = mg[i, pl.ds(j * 16, 16)] + efk * wm[j]
                    g = mg[i, pl.ds(d + j * 16, 16)] + efk * wg[j]
                    r[i, pl.ds(j * 16, 16)] = m / (1.0 + jnp.exp(-g))
                return carry

            lax.fori_loop(0, k, edge, 0)

        if not _PIPELINED:
            def chunk(c, carry):
                e0 = base + c * k
                pltpu.sync_copy(src_hbm.at[pl.ds(e0, k)], s_a)
                pltpu.sync_copy(ef_hbm.at[pl.ds(e0, k)], e_a)
                pltpu.sync_copy(dst_hbm.at[pl.ds(e0, k)], d_a)
                pltpu.async_copy(mg_hbm.at[s_a], mg_a, gsem_a).wait()
                compute(e_a, mg_a, r_a)
                pltpu.sync_copy(r_a, acc.at[d_a], add=True)
                return carry

            lax.fori_loop(0, ch, chunk, 0)
        else:
            def fetch_se(c, sv, ev, isem):
                e0 = base + c * k
                pltpu.async_copy(src_hbm.at[pl.ds(e0, k)], sv, isem)
                pltpu.async_copy(ef_hbm.at[pl.ds(e0, k)], ev, isem)

            def wait_se(c, sv, ev, isem):
                e0 = base + c * k
                pltpu.make_async_copy(src_hbm.at[pl.ds(e0, k)], sv, isem).wait()
                pltpu.make_async_copy(ef_hbm.at[pl.ds(e0, k)], ev, isem).wait()

            def fetch_dst(c, dv, dsem):
                pltpu.async_copy(dst_hbm.at[pl.ds(base + c * k, k)], dv, dsem)

            def wait_dst(c, dv, dsem):
                pltpu.make_async_copy(
                    dst_hbm.at[pl.ds(base + c * k, k)], dv, dsem).wait()

            def wait_gather(sv, mg, gsem):
                pltpu.make_async_copy(mg_hbm.at[sv], mg, gsem).wait()

            def wait_scatter(r, dv, ssem):
                pltpu.make_async_copy(r, acc.at[dv], ssem).wait()

            fetch_se(0, s_a, e_a, isem_a)
            fetch_se(1, s_b, e_b, isem_b)
            fetch_dst(0, d_a, dsem_a)
            fetch_dst(1, d_b, dsem_b)
            wait_se(0, s_a, e_a, isem_a)
            pltpu.async_copy(mg_hbm.at[s_a], mg_a, gsem_a)

            wait_gather(s_a, mg_a, gsem_a)
            wait_se(1, s_b, e_b, isem_b)
            pltpu.async_copy(mg_hbm.at[s_b], mg_b, gsem_b)
            compute(e_a, mg_a, r_a)
            wait_dst(0, d_a, dsem_a)
            pltpu.async_copy(r_a, acc.at[d_a], ssem_a, add=True)
            fetch_se(2, s_a, e_a, isem_a)
            wait_gather(s_b, mg_b, gsem_b)
            wait_se(2, s_a, e_a, isem_a)
            pltpu.async_copy(mg_hbm.at[s_a], mg_a, gsem_a)
            compute(e_b, mg_b, r_b)
            wait_dst(1, d_b, dsem_b)
            pltpu.async_copy(r_b, acc.at[d_b], ssem_b, add=True)
            fetch_se(3, s_b, e_b, isem_b)

            def step(c, sv, ev, dv, mg, r, gsem, ssem, isem, dsem,
                     sv2, ev2, mg2, gsem2, isem2):
                wait_gather(sv, mg, gsem)
                wait_se(c + 1, sv2, ev2, isem2)
                pltpu.async_copy(mg_hbm.at[sv2], mg2, gsem2)
                wait_scatter(r, dv, ssem)
                fetch_dst(c, dv, dsem)
                compute(ev, mg, r)
                wait_dst(c, dv, dsem)
                pltpu.async_copy(r, acc.at[dv], ssem, add=True)
                fetch_se(c + 2, sv, ev, isem)

            def pair(t, carry):
                c0 = 2 * t
                step(c0, s_a, e_a, d_a, mg_a, r_a, gsem_a, ssem_a, isem_a,
                     dsem_a, s_b, e_b, mg_b, gsem_b, isem_b)
                step(c0 + 1, s_b, e_b, d_b, mg_b, r_b, gsem_b, ssem_b, isem_b,
                     dsem_b, s_a, e_a, mg_a, gsem_a, isem_a)
                return carry

            lax.fori_loop(1, pairs - 1, pair, 0)

            wait_gather(s_a, mg_a, gsem_a)
            wait_se(ch - 1, s_b, e_b, isem_b)
            pltpu.async_copy(mg_hbm.at[s_b], mg_b, gsem_b)
            wait_scatter(r_a, d_a, ssem_a)
            fetch_dst(ch - 2, d_a, dsem_a)
            compute(e_a, mg_a, r_a)
            wait_dst(ch - 2, d_a, dsem_a)
            pltpu.async_copy(r_a, acc.at[d_a], ssem_a, add=True)
            wait_gather(s_b, mg_b, gsem_b)
            wait_scatter(r_b, d_b, ssem_b)
            fetch_dst(ch - 1, d_b, dsem_b)
            compute(e_b, mg_b, r_b)
            wait_dst(ch - 1, d_b, dsem_b)
            pltpu.async_copy(r_b, acc.at[d_b], ssem_b, add=True)

            wait_scatter(r_a, d_a, ssem_a)
            wait_scatter(r_b, d_b, ssem_b)

        plsc.subcore_barrier()

        def ocopy(t, carry):
            r0 = sid * rpt + t * k
            pltpu.sync_copy(acc.at[pl.ds(r0, k)], r_a)
            pltpu.sync_copy(r_a, out_hbm.at[cid, pl.ds(r0, k)])
            return carry

        lax.fori_loop(0, zc, ocopy, 0)

    return sc_edge


def kernel(x, edge_index, edge_feature, W_msg_node, b_msg_node, W_msg_edge, b_msg_edge,
           W_gate_node, b_gate_node, W_gate_edge, b_gate_edge, bias, W_self, b_self):
    n, d = x.shape
    e = edge_index.shape[1]
    k = 40
    src = edge_index[0].astype(jnp.int32)
    dst = edge_index[1].astype(jnp.int32)
    ef = edge_feature[:, 0]
    bm = (b_msg_node + b_msg_edge).reshape(1, d)
    bg = (b_gate_node + b_gate_edge).reshape(1, d)
    wmg = jnp.concatenate([W_msg_edge[0], W_gate_edge[0]])
    mg = _tc_mg(x, W_msg_node, W_gate_node, bm, bg)
    parts = _make_sc_edge(n, d, e, k)(mg, src, dst, ef, wmg)
    bf = (b_self + bias).reshape(1, d)
    return _tc_fin(parts, x, W_self, bf)


# speedup vs baseline: 5.6586x; 4.5132x over previous
import functools

import jax
import jax.numpy as jnp
from jax import lax
from jax.experimental import pallas as pl
from jax.experimental.pallas import tpu as pltpu
from jax.experimental.pallas import tpu_sc as plsc

_NC = 2
_NS = 16
_PIPELINED = True


def _mg_body(x_ref, wm_ref, wg_ref, bm_ref, bg_ref, mg_ref):
    d = x_ref.shape[1]
    xb = x_ref[...]
    mg_ref[:, :d] = jnp.dot(xb, wm_ref[...], preferred_element_type=jnp.float32) + bm_ref[...]
    mg_ref[:, d:] = jnp.dot(xb, wg_ref[...], preferred_element_type=jnp.float32) + bg_ref[...]


def _tc_mg(x, w_msg, w_gate, bm, bg):
    n, d = x.shape
    br = 1000
    return pl.pallas_call(
        _mg_body,
        grid=(n // br,),
        in_specs=[
            pl.BlockSpec((br, d), lambda i: (i, 0)),
            pl.BlockSpec((d, d), lambda i: (0, 0)),
            pl.BlockSpec((d, d), lambda i: (0, 0)),
            pl.BlockSpec((1, d), lambda i: (0, 0)),
            pl.BlockSpec((1, d), lambda i: (0, 0)),
        ],
        out_specs=pl.BlockSpec((br, 2 * d), lambda i: (i, 0)),
        out_shape=jax.ShapeDtypeStruct((n, 2 * d), jnp.float32),
    )(x, w_msg, w_gate, bm, bg)


def _fin_body(p_ref, x_ref, ws_ref, b_ref, o_ref):
    s = jnp.dot(x_ref[...], ws_ref[...], preferred_element_type=jnp.float32)
    o_ref[...] = jnp.maximum(p_ref[0] + p_ref[1] + s + b_ref[...], 0.0)


def _tc_fin(parts, x, w_self, bf):
    n, d = x.shape
    br = 1000
    return pl.pallas_call(
        _fin_body,
        grid=(n // br,),
        in_specs=[
            pl.BlockSpec((2, br, d), lambda i: (0, i, 0)),
            pl.BlockSpec((br, d), lambda i: (i, 0)),
            pl.BlockSpec((d, d), lambda i: (0, 0)),
            pl.BlockSpec((1, d), lambda i: (0, 0)),
        ],
        out_specs=pl.BlockSpec((br, d), lambda i: (i, 0)),
        out_shape=jax.ShapeDtypeStruct((n, d), jnp.float32),
    )(parts, x, w_self, bf)


@functools.lru_cache(maxsize=None)
def _make_sc_edge(n, d, e, k):
    nw = _NC * _NS
    epw = e // nw
    ch = epw // k
    rpt = -(-n // (_NS * k)) * k
    npad = rpt * _NS
    zc = rpt // k
    assert epw * nw == e and ch * k == epw and ch % 2 == 0 and ch >= 6
    assert k % 8 == 0 and k <= 128
    nj = d // 16
    pairs = ch // 2
    mesh = plsc.VectorSubcoreMesh(core_axis_name="c", subcore_axis_name="s")

    @functools.partial(
        pl.kernel,
        mesh=mesh,
        compiler_params=pltpu.CompilerParams(needs_layout_passes=False),
        out_type=jax.ShapeDtypeStruct((_NC, npad, d), jnp.float32),
        scratch_types=[
            pltpu.VMEM((k,), jnp.int32),
            pltpu.VMEM((k,), jnp.int32),
            pltpu.VMEM((k,), jnp.float32),
            pltpu.VMEM((k,), jnp.float32),
            pltpu.VMEM((k,), jnp.int32),
            pltpu.VMEM((k,), jnp.int32),
            pltpu.VMEM((k, 2 * d), jnp.float32),
            pltpu.VMEM((k, 2 * d), jnp.float32),
            pltpu.VMEM((k, d), jnp.float32),
            pltpu.VMEM((k, d), jnp.float32),
            pltpu.VMEM((2 * d,), jnp.float32),
            pltpu.VMEM_SHARED((npad, d), jnp.float32),
            [pltpu.SemaphoreType.DMA] * 8,
        ],
    )
    def sc_edge(mg_hbm, src_hbm, dst_hbm, ef_hbm, wmg_hbm, out_hbm,
                s_a, s_b, e_a, e_b, d_a, d_b, mg_a, mg_b, r_a, r_b,
                wmg_v, acc, sems):
        gsem_a, gsem_b, ssem_a, ssem_b, isem_a, isem_b, dsem_a, dsem_b = sems
        cid = lax.axis_index("c")
        sid = lax.axis_index("s")
        wid = sid * _NC + cid
        zeros16 = jnp.zeros((16,), jnp.float32)

        def zrow(i, carry):
            for j in range(nj):
                r_a[i, pl.ds(j * 16, 16)] = zeros16
            return carry

        lax.fori_loop(0, k, zrow, 0)
        row0 = sid * rpt

        def zcopy(t, carry):
            pltpu.sync_copy(r_a, acc.at[pl.ds(row0 + t * k, k)])
            return carry

        lax.fori_loop(0, zc, zcopy, 0)

        pltpu.sync_copy(wmg_hbm, wmg_v)
        wm = [wmg_v[pl.ds(j * 16, 16)] for j in range(nj)]
        wg = [wmg_v[pl.ds(d + j * 16, 16)] for j in range(nj)]
        plsc.subcore_barrier()

        base = wid * epw

        def compute(ev, mg, r):
            @plsc.parallel_loop(0, k, unroll=4)
            def edge(i):
                efk = plsc.load_gather(ev, [lax.broadcast(i, (16,))])
                for j in range(nj):
                    m = mg[i, pl.ds(j * 16, 16)] + efk * wm[j]
                    g = mg[i, pl.ds(d + j * 16, 16)] + efk * wg[j]
                    r[i, pl.ds(j * 16, 16)] = m / (1.0 + jnp.exp(-g))

        if not _PIPELINED:
            def chunk(c, carry):
                e0 = base + c * k
                pltpu.sync_copy(src_hbm.at[pl.ds(e0, k)], s_a)
                pltpu.sync_copy(ef_hbm.at[pl.ds(e0, k)], e_a)
                pltpu.sync_copy(dst_hbm.at[pl.ds(e0, k)], d_a)
                pltpu.async_copy(mg_hbm.at[s_a], mg_a, gsem_a).wait()
                compute(e_a, mg_a, r_a)
                pltpu.sync_copy(r_a, acc.at[d_a], add=True)
                return carry

            lax.fori_loop(0, ch, chunk, 0)
        else:
            def fetch_se(c, sv, ev, isem):
                e0 = base + c * k
                pltpu.async_copy(src_hbm.at[pl.ds(e0, k)], sv, isem)
                pltpu.async_copy(ef_hbm.at[pl.ds(e0, k)], ev, isem)

            def wait_se(c, sv, ev, isem):
                e0 = base + c * k
                pltpu.make_async_copy(src_hbm.at[pl.ds(e0, k)], sv, isem).wait()
                pltpu.make_async_copy(ef_hbm.at[pl.ds(e0, k)], ev, isem).wait()

            def fetch_dst(c, dv, dsem):
                pltpu.async_copy(dst_hbm.at[pl.ds(base + c * k, k)], dv, dsem)

            def wait_dst(c, dv, dsem):
                pltpu.make_async_copy(
                    dst_hbm.at[pl.ds(base + c * k, k)], dv, dsem).wait()

            def wait_gather(sv, mg, gsem):
                pltpu.make_async_copy(mg_hbm.at[sv], mg, gsem).wait()

            def wait_scatter(r, dv, ssem):
                pltpu.make_async_copy(r, acc.at[dv], ssem).wait()

            fetch_se(0, s_a, e_a, isem_a)
            fetch_se(1, s_b, e_b, isem_b)
            fetch_dst(0, d_a, dsem_a)
            fetch_dst(1, d_b, dsem_b)
            wait_se(0, s_a, e_a, isem_a)
            pltpu.async_copy(mg_hbm.at[s_a], mg_a, gsem_a)

            wait_gather(s_a, mg_a, gsem_a)
            wait_se(1, s_b, e_b, isem_b)
            pltpu.async_copy(mg_hbm.at[s_b], mg_b, gsem_b)
            compute(e_a, mg_a, r_a)
            wait_dst(0, d_a, dsem_a)
            pltpu.async_copy(r_a, acc.at[d_a], ssem_a, add=True)
            fetch_se(2, s_a, e_a, isem_a)
            wait_gather(s_b, mg_b, gsem_b)
            wait_se(2, s_a, e_a, isem_a)
            pltpu.async_copy(mg_hbm.at[s_a], mg_a, gsem_a)
            compute(e_b, mg_b, r_b)
            wait_dst(1, d_b, dsem_b)
            pltpu.async_copy(r_b, acc.at[d_b], ssem_b, add=True)
            fetch_se(3, s_b, e_b, isem_b)

            def step(c, sv, ev, dv, mg, r, gsem, ssem, isem, dsem,
                     sv2, ev2, mg2, gsem2, isem2):
                wait_gather(sv, mg, gsem)
                wait_se(c + 1, sv2, ev2, isem2)
                pltpu.async_copy(mg_hbm.at[sv2], mg2, gsem2)
                wait_scatter(r, dv, ssem)
                fetch_dst(c, dv, dsem)
                compute(ev, mg, r)
                wait_dst(c, dv, dsem)
                pltpu.async_copy(r, acc.at[dv], ssem, add=True)
                fetch_se(c + 2, sv, ev, isem)

            def pair(t, carry):
                c0 = 2 * t
                step(c0, s_a, e_a, d_a, mg_a, r_a, gsem_a, ssem_a, isem_a,
                     dsem_a, s_b, e_b, mg_b, gsem_b, isem_b)
                step(c0 + 1, s_b, e_b, d_b, mg_b, r_b, gsem_b, ssem_b, isem_b,
                     dsem_b, s_a, e_a, mg_a, gsem_a, isem_a)
                return carry

            lax.fori_loop(1, pairs - 1, pair, 0)

            wait_gather(s_a, mg_a, gsem_a)
            wait_se(ch - 1, s_b, e_b, isem_b)
            pltpu.async_copy(mg_hbm.at[s_b], mg_b, gsem_b)
            wait_scatter(r_a, d_a, ssem_a)
            fetch_dst(ch - 2, d_a, dsem_a)
            compute(e_a, mg_a, r_a)
            wait_dst(ch - 2, d_a, dsem_a)
            pltpu.async_copy(r_a, acc.at[d_a], ssem_a, add=True)
            wait_gather(s_b, mg_b, gsem_b)
            wait_scatter(r_b, d_b, ssem_b)
            fetch_dst(ch - 1, d_b, dsem_b)
            compute(e_b, mg_b, r_b)
            wait_dst(ch - 1, d_b, dsem_b)
            pltpu.async_copy(r_b, acc.at[d_b], ssem_b, add=True)

            wait_scatter(r_a, d_a, ssem_a)
            wait_scatter(r_b, d_b, ssem_b)

        plsc.subcore_barrier()

        def ocopy(t, carry):
            r0 = sid * rpt + t * k
            pltpu.sync_copy(acc.at[pl.ds(r0, k)], r_a)
            pltpu.sync_copy(r_a, out_hbm.at[cid, pl.ds(r0, k)])
            return carry

        lax.fori_loop(0, zc, ocopy, 0)

    return sc_edge


def kernel(x, edge_index, edge_feature, W_msg_node, b_msg_node, W_msg_edge, b_msg_edge,
           W_gate_node, b_gate_node, W_gate_edge, b_gate_edge, bias, W_self, b_self):
    n, d = x.shape
    e = edge_index.shape[1]
    k = 40
    src = edge_index[0].astype(jnp.int32)
    dst = edge_index[1].astype(jnp.int32)
    ef = edge_feature[:, 0]
    bm = (b_msg_node + b_msg_edge).reshape(1, d)
    bg = (b_gate_node + b_gate_edge).reshape(1, d)
    wmg = jnp.concatenate([W_msg_edge[0], W_gate_edge[0]])
    mg = _tc_mg(x, W_msg_node, W_gate_node, bm, bg)
    parts = _make_sc_edge(n, d, e, k)(mg, src, dst, ef, wmg)
    bf = (b_self + bias).reshape(1, d)
    return _tc_fin(parts, x, W_self, bf)
